# baseline (speedup 1.0000x reference)
import functools

import jax
import jax.numpy as jnp
from jax import lax
from jax.experimental import pallas as pl
from jax.experimental.pallas import tpu as pltpu
from jax.experimental.pallas import tpu_sc as plsc

B = 4
S = 4096
D = 2048
K = S // 2
ROWS_PER_BATCH = K + 1
BATCH_STRIDE = 2112
ROWS_PAD = B * BATCH_STRIDE
NC, NS = 2, 16
NW = NC * NS
RPW = ROWS_PAD // NW
CH = 24
NCHUNK = RPW // CH


def _gather_body(x_hbm, gidx_hbm, out_hbm, idx_v, buf0, buf1, sem0, sem1):
    wid = lax.axis_index("s") * NC + lax.axis_index("c")
    base = wid * RPW
    pltpu.sync_copy(gidx_hbm.at[pl.ds(base, RPW)], idx_v)
    bufs = (buf0, buf1)
    sems = (sem0, sem1)
    copies = [None, None]
    copies[0] = pltpu.async_copy(x_hbm.at[idx_v.at[pl.ds(0, CH)]], bufs[0], sems[0])
    for c in range(NCHUNK):
        if c + 1 < NCHUNK:
            copies[(c + 1) % 2] = pltpu.async_copy(
                x_hbm.at[idx_v.at[pl.ds((c + 1) * CH, CH)]],
                bufs[(c + 1) % 2], sems[(c + 1) % 2])
        copies[c % 2].wait()
        pltpu.sync_copy(bufs[c % 2], out_hbm.at[pl.ds(base + c * CH, CH)])


_gather = functools.partial(
    pl.kernel,
    out_type=jax.ShapeDtypeStruct((ROWS_PAD, D), jnp.float32),
    mesh=plsc.VectorSubcoreMesh(core_axis_name="c", subcore_axis_name="s"),
    scratch_types=[
        pltpu.VMEM((RPW,), jnp.int32),
        pltpu.VMEM((CH, D), jnp.float32),
        pltpu.VMEM((CH, D), jnp.float32),
        pltpu.SemaphoreType.DMA,
        pltpu.SemaphoreType.DMA,
    ],
)(_gather_body)


def kernel(x, mask):
    _, idx = jax.lax.top_k(mask, K)
    rows = jnp.concatenate(
        [jnp.zeros((B, 1), jnp.int32), idx.astype(jnp.int32) + 1], axis=1)
    rows = rows + (S + 1) * jnp.arange(B, dtype=jnp.int32)[:, None]
    gidx = jnp.pad(rows, ((0, 0), (0, BATCH_STRIDE - ROWS_PER_BATCH)))
    x_flat = x.reshape(B * (S + 1), D)
    out_pad = _gather(x_flat, gidx.reshape(-1))
    return out_pad.reshape(B, BATCH_STRIDE, D)[:, :ROWS_PER_BATCH, :]


# speedup vs baseline: 1.5038x; 1.5038x over previous
import functools

import jax
import jax.numpy as jnp
from jax import lax
from jax.experimental import pallas as pl
from jax.experimental.pallas import tpu as pltpu
from jax.experimental.pallas import tpu_sc as plsc

B = 4
S = 4096
D = 2048
K = S // 2
OUT_ROWS = K + 1
IDX_STRIDE = 2056
NC, NS = 2, 16
WPB = 8
RPW = K // WPB
CH = 16
NCHUNK = RPW // CH


def _gather_body(x_hbm, lidx_hbm, out_hbm, idx_v, buf0, buf1, buf_t, sem0,
                 sem1):
    w = lax.axis_index("s") * NC + lax.axis_index("c")
    b = w // WPB
    j = w % WPB
    base = j * RPW
    pltpu.sync_copy(lidx_hbm.at[pl.ds(b * IDX_STRIDE + base, RPW)], idx_v)
    bufs = (buf0, buf1)
    sems = (sem0, sem1)
    copies = [None, None]

    def start(ci, slot):
        copies[slot] = pltpu.async_copy(
            x_hbm.at[b].at[idx_v.at[pl.ds(ci * CH, CH)]],
            bufs[slot], sems[slot])

    start(0, 0)
    for ci in range(NCHUNK):
        if ci + 1 < NCHUNK:
            start(ci + 1, (ci + 1) % 2)
        copies[ci % 2].wait()
        pltpu.sync_copy(bufs[ci % 2],
                        out_hbm.at[b].at[pl.ds(base + ci * CH, CH)])

    @pl.when(j == WPB - 1)
    def _():
        pltpu.sync_copy(lidx_hbm.at[pl.ds(b * IDX_STRIDE + K, 8)],
                        idx_v.at[pl.ds(0, 8)])
        pltpu.async_copy(
            x_hbm.at[b].at[idx_v.at[pl.ds(0, 1)]], buf_t, sem0).wait()
        pltpu.sync_copy(buf_t, out_hbm.at[b].at[pl.ds(K, 1)])


_gather = functools.partial(
    pl.kernel,
    out_type=jax.ShapeDtypeStruct((B, OUT_ROWS, D), jnp.float32),
    mesh=plsc.VectorSubcoreMesh(core_axis_name="c", subcore_axis_name="s"),
    scratch_types=[
        pltpu.VMEM((RPW,), jnp.int32),
        pltpu.VMEM((CH, D), jnp.float32),
        pltpu.VMEM((CH, D), jnp.float32),
        pltpu.VMEM((1, D), jnp.float32),
        pltpu.SemaphoreType.DMA,
        pltpu.SemaphoreType.DMA,
    ],
)(_gather_body)


def kernel(x, mask):
    _, idx = jax.lax.top_k(mask, K)
    rows = jnp.concatenate(
        [jnp.zeros((B, 1), jnp.int32), idx.astype(jnp.int32) + 1], axis=1)
    lidx = jnp.pad(rows, ((0, 0), (0, IDX_STRIDE - OUT_ROWS)))
    return _gather(x, lidx.reshape(-1))
